# Initial kernel scaffold; baseline (speedup 1.0000x reference)
import functools

import jax
import jax.numpy as jnp
from jax import lax
from jax.experimental import pallas as pl
from jax.experimental.pallas import tpu as pltpu
from jax.experimental.pallas import tpu_sc as plsc

_NC = 2
_NS = 16
_NW = _NC * _NS
_L = 16
_D = 128
_CHUNK = 128
_ROW_BLK = 512


def _round_up(a, b):
    return (a + b - 1) // b * b


@functools.lru_cache(None)
def _deg_kernel(n_pad, n_chunks):
    rows_pt = n_pad // _NS
    mesh = plsc.VectorSubcoreMesh(core_axis_name="c", subcore_axis_name="s")

    @functools.partial(
        pl.kernel,
        out_type=jax.ShapeDtypeStruct((_NC, n_pad, _D), jnp.float32),
        mesh=mesh,
        scratch_types=[
            pltpu.VMEM((n_chunks, _CHUNK), jnp.int32),
            pltpu.VMEM((_CHUNK, _L), jnp.float32),
            pltpu.VMEM((rows_pt, _L), jnp.float32),
            pltpu.VMEM((rows_pt, _D), jnp.float32),
            pltpu.VMEM_SHARED((n_pad, _L), jnp.float32),
        ],
    )
    def deg(dst_hbm, out_hbm, dstv, onesv, locv, outv, acc_sh):
        cid = lax.axis_index("c")
        sid = lax.axis_index("s")
        wid = sid * _NC + cid
        base = sid * rows_pt
        ones16 = jnp.ones((_L,), jnp.float32)
        zeros16 = jnp.zeros((_L,), jnp.float32)

        def fill_ones(r, carry):
            onesv[r, :] = ones16
            return carry

        lax.fori_loop(0, _CHUNK, fill_ones, 0)

        def fill_zero(r, carry):
            locv[r, :] = zeros16
            return carry

        lax.fori_loop(0, rows_pt, fill_zero, 0)
        pltpu.sync_copy(locv, acc_sh.at[pl.ds(base, rows_pt)])
        plsc.subcore_barrier()

        pltpu.sync_copy(dst_hbm.at[wid], dstv)

        def chunk(j, carry):
            pltpu.sync_copy(onesv, acc_sh.at[dstv.at[j]], add=True)
            return carry

        lax.fori_loop(0, n_chunks, chunk, 0)
        plsc.subcore_barrier()

        pltpu.sync_copy(acc_sh.at[pl.ds(base, rows_pt)], locv)

        def widen(i, carry):
            v = locv[i, :]
            for k in range(_D // _L):
                outv[i, pl.ds(k * _L, _L)] = v
            return carry

        lax.fori_loop(0, rows_pt, widen, 0)
        pltpu.sync_copy(outv, out_hbm.at[cid, pl.ds(base, rows_pt)])

    return deg


@functools.lru_cache(None)
def _edge_kernel(n_pad, n_chunks):
    rows_pt = n_pad // _NS
    n_cp = rows_pt // _CHUNK
    mesh = plsc.VectorSubcoreMesh(core_axis_name="c", subcore_axis_name="s")

    @functools.partial(
        pl.kernel,
        out_type=jax.ShapeDtypeStruct((_NC, n_pad, _D), jnp.float32),
        mesh=mesh,
        scratch_types=[
            pltpu.VMEM((n_chunks, _CHUNK), jnp.int32),
            pltpu.VMEM((n_chunks, _CHUNK), jnp.int32),
            pltpu.VMEM((2, _CHUNK, _D), jnp.float32),
            pltpu.VMEM_SHARED((n_pad, _D), jnp.float32),
            pltpu.SemaphoreType.DMA,
            pltpu.SemaphoreType.DMA,
        ],
    )
    def edge(src_hbm, dst_hbm, hp_hbm, out_hbm, srcv, dstv, rows, acc_sh,
             sem0, sem1):
        cid = lax.axis_index("c")
        sid = lax.axis_index("s")
        wid = sid * _NC + cid
        base = sid * rows_pt
        zeros16 = jnp.zeros((_L,), jnp.float32)

        def zrow(r, carry):
            for k in range(_D // _L):
                rows[0, r, pl.ds(k * _L, _L)] = zeros16
            return carry

        lax.fori_loop(0, _CHUNK, zrow, 0)
        for t in range(n_cp):
            pltpu.sync_copy(rows.at[0],
                            acc_sh.at[pl.ds(base + t * _CHUNK, _CHUNK)])
        plsc.subcore_barrier()

        pltpu.sync_copy(src_hbm.at[wid], srcv)
        pltpu.sync_copy(dst_hbm.at[wid], dstv)

        sems = (sem0, sem1)
        pltpu.async_copy(hp_hbm.at[srcv.at[0]], rows.at[0], sem0)

        def outer(jj, carry):
            for b in range(2):
                j = jj * 2 + b

                @pl.when(j + 1 < n_chunks)
                def _start():
                    pltpu.async_copy(hp_hbm.at[srcv.at[j + 1]],
                                     rows.at[1 - b], sems[1 - b])

                pltpu.make_async_copy(hp_hbm.at[srcv.at[j]], rows.at[b],
                                      sems[b]).wait()
                pltpu.sync_copy(rows.at[b], acc_sh.at[dstv.at[j]], add=True)
            return carry

        lax.fori_loop(0, n_chunks // 2, outer, 0)
        plsc.subcore_barrier()

        for t in range(n_cp):
            pltpu.sync_copy(acc_sh.at[pl.ds(base + t * _CHUNK, _CHUNK)],
                            out_hbm.at[cid, pl.ds(base + t * _CHUNK, _CHUNK)])

    return edge


def _mm1_body(x_ref, w_ref, d0_ref, d1_ref, hp_ref, dinv_ref):
    dinv = lax.rsqrt(d0_ref[...] + d1_ref[...] + 1.0)
    h = jnp.dot(x_ref[...], w_ref[...], preferred_element_type=jnp.float32)
    hp_ref[...] = dinv * h
    dinv_ref[...] = dinv


def _mm2_body(x_ref, w_ref, dinv_ref, hp_ref):
    hp_ref[...] = dinv_ref[...] * jnp.dot(
        x_ref[...], w_ref[...], preferred_element_type=jnp.float32)


def _post_body(a0_ref, a1_ref, hp_ref, dinv_ref, b_ref, g_ref, bb_ref,
               out_ref, *, final):
    s = dinv_ref[...] * (a0_ref[...] + a1_ref[...] + hp_ref[...]) + b_ref[...]
    mu = jnp.mean(s, axis=-1, keepdims=True)
    c = s - mu
    var = jnp.mean(c * c, axis=-1, keepdims=True)
    t = c * lax.rsqrt(var + 1e-5) * g_ref[...] + bb_ref[...]
    e = jnp.where(t > 0, t, jnp.expm1(t))
    if final:
        m = jnp.max(e, axis=-1, keepdims=True)
        lse = jnp.log(jnp.sum(jnp.exp(e - m), axis=-1, keepdims=True)) + m
        e = e - lse
    out_ref[...] = e


def _row_spec():
    return pl.BlockSpec((_ROW_BLK, _D), lambda i: (i, 0))


def _full_spec():
    return pl.BlockSpec((_D, _D), lambda i: (0, 0))


def _vec_spec():
    return pl.BlockSpec((1, _D), lambda i: (0, 0))


@functools.lru_cache(None)
def _mm1_call(n_pad):
    return pl.pallas_call(
        _mm1_body,
        grid=(n_pad // _ROW_BLK,),
        in_specs=[_row_spec(), _full_spec(), _row_spec(), _row_spec()],
        out_specs=[_row_spec(), _row_spec()],
        out_shape=[jax.ShapeDtypeStruct((n_pad, _D), jnp.float32),
                   jax.ShapeDtypeStruct((n_pad, _D), jnp.float32)],
    )


@functools.lru_cache(None)
def _mm2_call(n_pad):
    return pl.pallas_call(
        _mm2_body,
        grid=(n_pad // _ROW_BLK,),
        in_specs=[_row_spec(), _full_spec(), _row_spec()],
        out_specs=_row_spec(),
        out_shape=jax.ShapeDtypeStruct((n_pad, _D), jnp.float32),
    )


@functools.lru_cache(None)
def _post_call(n_pad, final):
    return pl.pallas_call(
        functools.partial(_post_body, final=final),
        grid=(n_pad // _ROW_BLK,),
        in_specs=[_row_spec(), _row_spec(), _row_spec(), _row_spec(),
                  _vec_spec(), _vec_spec(), _vec_spec()],
        out_specs=_row_spec(),
        out_shape=jax.ShapeDtypeStruct((n_pad, _D), jnp.float32),
    )


def kernel(x, edge_index, W1, b1, ln1_g, ln1_b, W2, b2, ln2_g, ln2_b):
    n, d = x.shape
    e = edge_index.shape[1]
    n_pad = _round_up(n, _NS * _CHUNK)
    epw = _round_up(e, _NW * _CHUNK * 2) // _NW
    n_chunks = epw // _CHUNK
    e_pad = epw * _NW
    pad = e_pad - e

    padv = jnp.full((pad,), n, jnp.int32)
    src = jnp.concatenate([edge_index[0], padv]).reshape(_NW, n_chunks, _CHUNK)
    dst = jnp.concatenate([edge_index[1], padv]).reshape(_NW, n_chunks, _CHUNK)
    x_pad = jnp.concatenate([x, jnp.zeros((n_pad - n, d), x.dtype)])

    deg = _deg_kernel(n_pad, n_chunks)(dst)

    b1r = b1.reshape(1, _D)
    g1r = ln1_g.reshape(1, _D)
    bb1r = ln1_b.reshape(1, _D)
    b2r = b2.reshape(1, _D)
    g2r = ln2_g.reshape(1, _D)
    bb2r = ln2_b.reshape(1, _D)

    edge_call = _edge_kernel(n_pad, n_chunks)

    hp1, dinvw = _mm1_call(n_pad)(x_pad, W1, deg[0], deg[1])
    acc1 = edge_call(src, dst, hp1)
    z1 = _post_call(n_pad, False)(acc1[0], acc1[1], hp1, dinvw,
                                  b1r, g1r, bb1r)
    hp2 = _mm2_call(n_pad)(z1, W2, dinvw)
    acc2 = edge_call(src, dst, hp2)
    z2 = _post_call(n_pad, True)(acc2[0], acc2[1], hp2, dinvw,
                                 b2r, g2r, bb2r)
    return z2[:n]


# speedup vs baseline: 15.2851x; 15.2851x over previous
import functools

import jax
import jax.numpy as jnp
from jax import lax
from jax.experimental import pallas as pl
from jax.experimental.pallas import tpu as pltpu
from jax.experimental.pallas import tpu_sc as plsc

_NC = 2
_NS = 16
_L = 16
_D = 128
_H = _D // 2
_CHUNK = 128
_ROW_BLK = 512

_SC_PARAMS = pltpu.CompilerParams(use_tc_tiling_on_sc=False)


def _round_up(a, b):
    return (a + b - 1) // b * b


@functools.lru_cache(None)
def _deg_kernel(n_pad, n_chunks):
    rows_pt = n_pad // _NS
    n_cp = rows_pt // _CHUNK
    hc = n_chunks // 2
    mesh = plsc.VectorSubcoreMesh(core_axis_name="c", subcore_axis_name="s")

    @functools.partial(
        pl.kernel,
        out_type=jax.ShapeDtypeStruct((2 * n_pad, _D), jnp.float32),
        mesh=mesh,
        scratch_types=[
            pltpu.VMEM((n_chunks, _CHUNK), jnp.int32),
            pltpu.VMEM((_CHUNK, _L), jnp.float32),
            pltpu.VMEM((rows_pt, _L), jnp.float32),
            pltpu.VMEM((_CHUNK, _D), jnp.float32),
            pltpu.VMEM_SHARED((n_pad, _L), jnp.float32),
        ],
        compiler_params=_SC_PARAMS,
    )
    def deg(dst_hbm, out_hbm, dstv, onesv, locv, outv, acc_sh):
        cid = lax.axis_index("c")
        sid = lax.axis_index("s")
        base = sid * rows_pt
        out_base = cid * n_pad + base
        ones16 = jnp.ones((_L,), jnp.float32)
        zeros16 = jnp.zeros((_L,), jnp.float32)

        def fill_ones(r, carry):
            onesv[r, :] = ones16
            return carry

        lax.fori_loop(0, _CHUNK, fill_ones, 0)

        def fill_zero(r, carry):
            locv[r, :] = zeros16
            return carry

        lax.fori_loop(0, rows_pt, fill_zero, 0)
        pltpu.sync_copy(locv, acc_sh.at[pl.ds(base, rows_pt)])
        plsc.subcore_barrier()

        pltpu.sync_copy(dst_hbm.at[sid], dstv)

        def chunk(jj, carry):
            pltpu.sync_copy(onesv, acc_sh.at[dstv.at[cid * hc + jj]],
                            add=True)
            return carry

        lax.fori_loop(0, hc, chunk, 0)
        plsc.subcore_barrier()

        pltpu.sync_copy(acc_sh.at[pl.ds(base, rows_pt)], locv)

        for t in range(n_cp):
            def widen(i, carry):
                v = locv[t * _CHUNK + i, :]
                for k in range(_D // _L):
                    outv[i, pl.ds(k * _L, _L)] = v
                return carry

            lax.fori_loop(0, _CHUNK, widen, 0)
            pltpu.sync_copy(outv,
                            out_hbm.at[pl.ds(out_base + t * _CHUNK, _CHUNK)])

    return deg


@functools.lru_cache(None)
def _edge_kernel(n_pad, n_chunks):
    rows_pt = n_pad // _NS
    n_cp = rows_pt // _CHUNK
    mesh = plsc.VectorSubcoreMesh(core_axis_name="c", subcore_axis_name="s")

    @functools.partial(
        pl.kernel,
        out_type=jax.ShapeDtypeStruct((2 * n_pad, _H), jnp.float32),
        mesh=mesh,
        scratch_types=[
            pltpu.VMEM((n_chunks, _CHUNK), jnp.int32),
            pltpu.VMEM((n_chunks, _CHUNK), jnp.int32),
            pltpu.VMEM((2, _CHUNK, _H), jnp.float32),
            pltpu.VMEM_SHARED((n_pad, _H), jnp.float32),
            pltpu.SemaphoreType.DMA,
            pltpu.SemaphoreType.DMA,
        ],
        compiler_params=_SC_PARAMS,
    )
    def edge(src_hbm, dst_hbm, hp_hbm, out_hbm, srcv, dstv, rows, acc_sh,
             sem0, sem1):
        cid = lax.axis_index("c")
        sid = lax.axis_index("s")
        base = sid * rows_pt
        out_base = cid * n_pad + base
        zeros16 = jnp.zeros((_L,), jnp.float32)

        def zrow(r, carry):
            for k in range(_H // _L):
                rows[0, r, pl.ds(k * _L, _L)] = zeros16
            return carry

        lax.fori_loop(0, _CHUNK, zrow, 0)
        for t in range(n_cp):
            pltpu.sync_copy(rows.at[0],
                            acc_sh.at[pl.ds(base + t * _CHUNK, _CHUNK)])
        plsc.subcore_barrier()

        pltpu.sync_copy(src_hbm.at[sid], srcv)
        pltpu.sync_copy(dst_hbm.at[sid], dstv)

        def addoff(j, carry):
            for k in range(_CHUNK // _L):
                sl = pl.ds(k * _L, _L)
                srcv[j, sl] = srcv[j, sl] * 2 + cid
            return carry

        lax.fori_loop(0, n_chunks, addoff, 0)

        sems = (sem0, sem1)
        pltpu.async_copy(hp_hbm.at[srcv.at[0]], rows.at[0], sem0)

        def outer(jj, carry):
            for b in range(2):
                j = jj * 2 + b

                @pl.when(j + 1 < n_chunks)
                def _start():
                    pltpu.async_copy(hp_hbm.at[srcv.at[j + 1]],
                                     rows.at[1 - b], sems[1 - b])

                pltpu.make_async_copy(hp_hbm.at[srcv.at[j]], rows.at[b],
                                      sems[b]).wait()
                pltpu.sync_copy(rows.at[b], acc_sh.at[dstv.at[j]], add=True)
            return carry

        lax.fori_loop(0, n_chunks // 2, outer, 0)
        plsc.subcore_barrier()

        for t in range(n_cp):
            pltpu.sync_copy(acc_sh.at[pl.ds(base + t * _CHUNK, _CHUNK)],
                            out_hbm.at[pl.ds(out_base + t * _CHUNK, _CHUNK)])

    return edge


def _mm1_body(x_ref, w_ref, d0_ref, d1_ref, hp_ref, dinv_ref):
    dinv = lax.rsqrt(d0_ref[...] + d1_ref[...] + 1.0)
    h = jnp.dot(x_ref[...], w_ref[...], preferred_element_type=jnp.float32)
    hp_ref[...] = dinv * h
    dinv_ref[...] = dinv


def _mm2_body(x_ref, w_ref, dinv_ref, hp_ref):
    hp_ref[...] = dinv_ref[...] * jnp.dot(
        x_ref[...], w_ref[...], preferred_element_type=jnp.float32)


def _post_body(al_ref, ah_ref, hp_ref, dv_ref, b_ref, g_ref, bb_ref,
               out_ref, *, final):
    dv = dv_ref[...]
    hpv = hp_ref[...]
    bv = b_ref[...]
    sl = dv[:, 0:_H] * (al_ref[...] + hpv[:, 0:_H]) + bv[:, 0:_H]
    sh = dv[:, _H:_D] * (ah_ref[...] + hpv[:, _H:_D]) + bv[:, _H:_D]
    mu = (jnp.sum(sl, -1, keepdims=True) +
          jnp.sum(sh, -1, keepdims=True)) * (1.0 / _D)
    cl = sl - mu
    ch = sh - mu
    var = (jnp.sum(cl * cl, -1, keepdims=True) +
           jnp.sum(ch * ch, -1, keepdims=True)) * (1.0 / _D)
    r = lax.rsqrt(var + 1e-5)
    gv = g_ref[...]
    bbv = bb_ref[...]
    tl = cl * r * gv[:, 0:_H] + bbv[:, 0:_H]
    th = ch * r * gv[:, _H:_D] + bbv[:, _H:_D]
    el = jnp.where(tl > 0, tl, jnp.exp(jnp.minimum(tl, 0.0)) - 1.0)
    eh = jnp.where(th > 0, th, jnp.exp(jnp.minimum(th, 0.0)) - 1.0)
    if final:
        m = jnp.maximum(jnp.max(el, -1, keepdims=True),
                        jnp.max(eh, -1, keepdims=True))
        lse = jnp.log(jnp.sum(jnp.exp(el - m), -1, keepdims=True) +
                      jnp.sum(jnp.exp(eh - m), -1, keepdims=True)) + m
        el = el - lse
        eh = eh - lse
    out_ref[:, 0:_H] = el
    out_ref[:, _H:_D] = eh


def _row_spec():
    return pl.BlockSpec((_ROW_BLK, _D), lambda i: (i, 0))


@functools.lru_cache(None)
def _mm1_call(n_pad):
    nb = n_pad // _ROW_BLK
    return pl.pallas_call(
        _mm1_body,
        grid=(nb,),
        in_specs=[
            _row_spec(),
            pl.BlockSpec((_D, _D), lambda i: (0, 0)),
            _row_spec(),
            pl.BlockSpec((_ROW_BLK, _D), lambda i: (nb + i, 0)),
        ],
        out_specs=[_row_spec(), _row_spec()],
        out_shape=[jax.ShapeDtypeStruct((n_pad, _D), jnp.float32),
                   jax.ShapeDtypeStruct((n_pad, _D), jnp.float32)],
    )


@functools.lru_cache(None)
def _mm2_call(n_pad):
    nb = n_pad // _ROW_BLK
    return pl.pallas_call(
        _mm2_body,
        grid=(nb,),
        in_specs=[
            _row_spec(),
            pl.BlockSpec((_D, _D), lambda i: (0, 0)),
            _row_spec(),
        ],
        out_specs=_row_spec(),
        out_shape=jax.ShapeDtypeStruct((n_pad, _D), jnp.float32),
    )


@functools.lru_cache(None)
def _post_call(n_pad, final):
    nb = n_pad // _ROW_BLK
    half = pl.BlockSpec((_ROW_BLK, _H), lambda i: (i, 0))
    halfhi = pl.BlockSpec((_ROW_BLK, _H), lambda i: (nb + i, 0))
    vec = pl.BlockSpec((1, _D), lambda i: (0, 0))
    return pl.pallas_call(
        functools.partial(_post_body, final=final),
        grid=(nb,),
        in_specs=[half, halfhi, _row_spec(), _row_spec(), vec, vec, vec],
        out_specs=pl.BlockSpec((_ROW_BLK, _D), lambda i: (i, 0)),
        out_shape=jax.ShapeDtypeStruct((n_pad, _D), jnp.float32),
    )


def kernel(x, edge_index, W1, b1, ln1_g, ln1_b, W2, b2, ln2_g, ln2_b):
    n, d = x.shape
    e = edge_index.shape[1]
    n_pad = _round_up(n, _NS * _CHUNK)
    e_pad = _round_up(e, _NS * _CHUNK * 2)
    n_chunks = e_pad // (_NS * _CHUNK)
    pad = e_pad - e

    padv = jnp.full((pad,), n, jnp.int32)
    src = jnp.concatenate([edge_index[0], padv]).reshape(_NS, n_chunks, _CHUNK)
    dst = jnp.concatenate([edge_index[1], padv]).reshape(_NS, n_chunks, _CHUNK)
    x_pad = jnp.concatenate([x, jnp.zeros((n_pad - n, d), x.dtype)])

    deg = _deg_kernel(n_pad, n_chunks)(dst)

    b1r = b1.reshape(1, _D)
    g1r = ln1_g.reshape(1, _D)
    bb1r = ln1_b.reshape(1, _D)
    b2r = b2.reshape(1, _D)
    g2r = ln2_g.reshape(1, _D)
    bb2r = ln2_b.reshape(1, _D)

    edge_call = _edge_kernel(n_pad, n_chunks)

    hp1, dinvw = _mm1_call(n_pad)(x_pad, W1, deg, deg)
    acc1 = edge_call(src, dst, hp1.reshape(2 * n_pad, _H))
    z1 = _post_call(n_pad, False)(acc1, acc1, hp1, dinvw, b1r, g1r, bb1r)
    hp2 = _mm2_call(n_pad)(z1, W2, dinvw)
    acc2 = edge_call(src, dst, hp2.reshape(2 * n_pad, _H))
    z2 = _post_call(n_pad, True)(acc2, acc2, hp2, dinvw, b2r, g2r, bb2r)
    return z2[:n]


# speedup vs baseline: 15.5816x; 1.0194x over previous
import functools

import jax
import jax.numpy as jnp
from jax import lax
from jax.experimental import pallas as pl
from jax.experimental.pallas import tpu as pltpu
from jax.experimental.pallas import tpu_sc as plsc

_NC = 2
_NS = 16
_L = 16
_D = 128
_H = _D // 2
_CHUNK = 128
_ROW_BLK = 512

_SC_PARAMS = pltpu.CompilerParams(use_tc_tiling_on_sc=False)


def _round_up(a, b):
    return (a + b - 1) // b * b


@functools.lru_cache(None)
def _deg_kernel(n_pad, n_chunks):
    rows_pt = n_pad // _NS
    n_cp = rows_pt // _CHUNK
    hc = n_chunks // 2
    mesh = plsc.VectorSubcoreMesh(core_axis_name="c", subcore_axis_name="s")

    @functools.partial(
        pl.kernel,
        out_type=jax.ShapeDtypeStruct((2 * n_pad, _D), jnp.float32),
        mesh=mesh,
        scratch_types=[
            pltpu.VMEM((n_chunks, _CHUNK), jnp.int32),
            pltpu.VMEM((_CHUNK, _L), jnp.float32),
            pltpu.VMEM((rows_pt, _L), jnp.float32),
            pltpu.VMEM((_CHUNK, _D), jnp.float32),
            pltpu.VMEM_SHARED((n_pad, _L), jnp.float32),
        ],
        compiler_params=_SC_PARAMS,
    )
    def deg(dst_hbm, out_hbm, dstv, onesv, locv, outv, acc_sh):
        cid = lax.axis_index("c")
        sid = lax.axis_index("s")
        base = sid * rows_pt
        out_base = cid * n_pad + base
        ones16 = jnp.ones((_L,), jnp.float32)
        zeros16 = jnp.zeros((_L,), jnp.float32)

        def fill_ones(r, carry):
            onesv[r, :] = ones16
            return carry

        lax.fori_loop(0, _CHUNK, fill_ones, 0)

        def fill_zero(r, carry):
            locv[r, :] = zeros16
            return carry

        lax.fori_loop(0, rows_pt, fill_zero, 0)
        pltpu.sync_copy(locv, acc_sh.at[pl.ds(base, rows_pt)])
        plsc.subcore_barrier()

        pltpu.sync_copy(dst_hbm.at[sid], dstv)

        def chunk(jj, carry):
            pltpu.sync_copy(onesv, acc_sh.at[dstv.at[cid * hc + jj]],
                            add=True)
            return carry

        lax.fori_loop(0, hc, chunk, 0)
        plsc.subcore_barrier()

        pltpu.sync_copy(acc_sh.at[pl.ds(base, rows_pt)], locv)

        for t in range(n_cp):
            def widen(i, carry):
                v = locv[t * _CHUNK + i, :]
                for k in range(_D // _L):
                    outv[i, pl.ds(k * _L, _L)] = v
                return carry

            lax.fori_loop(0, _CHUNK, widen, 0)
            pltpu.sync_copy(outv,
                            out_hbm.at[pl.ds(out_base + t * _CHUNK, _CHUNK)])

    return deg


@functools.lru_cache(None)
def _edge_kernel(n_pad, n_chunks):
    rows_pt = n_pad // _NS
    n_cp = rows_pt // _CHUNK
    mesh = plsc.VectorSubcoreMesh(core_axis_name="c", subcore_axis_name="s")

    @functools.partial(
        pl.kernel,
        out_type=jax.ShapeDtypeStruct((2 * n_pad, _H), jnp.float32),
        mesh=mesh,
        scratch_types=[
            pltpu.VMEM((n_chunks, _CHUNK), jnp.int32),
            pltpu.VMEM((n_chunks, _CHUNK), jnp.int32),
            pltpu.VMEM((2, _CHUNK, _H), jnp.float32),
            pltpu.VMEM_SHARED((n_pad, _H), jnp.float32),
            pltpu.SemaphoreType.DMA,
            pltpu.SemaphoreType.DMA,
        ],
        compiler_params=_SC_PARAMS,
    )
    def edge(src_hbm, dst_hbm, hp_hbm, out_hbm, srcv, dstv, rows, acc_sh,
             sem0, sem1):
        cid = lax.axis_index("c")
        sid = lax.axis_index("s")
        base = sid * rows_pt
        out_base = cid * n_pad + base
        zeros16 = jnp.zeros((_L,), jnp.float32)

        def zrow(r, carry):
            for k in range(_H // _L):
                rows[0, r, pl.ds(k * _L, _L)] = zeros16
            return carry

        lax.fori_loop(0, _CHUNK, zrow, 0)
        for t in range(n_cp):
            pltpu.sync_copy(rows.at[0],
                            acc_sh.at[pl.ds(base + t * _CHUNK, _CHUNK)])
        plsc.subcore_barrier()

        pltpu.sync_copy(src_hbm.at[sid], srcv)
        pltpu.sync_copy(dst_hbm.at[sid], dstv)

        def addoff(j, carry):
            for k in range(_CHUNK // _L):
                sl = pl.ds(k * _L, _L)
                srcv[j, sl] = srcv[j, sl] * 2 + cid
            return carry

        lax.fori_loop(0, n_chunks, addoff, 0)

        sems = (sem0, sem1)
        pltpu.async_copy(hp_hbm.at[srcv.at[0]], rows.at[0], sem0)

        def outer(jj, carry):
            for b in range(2):
                j = jj * 2 + b

                @pl.when(j + 1 < n_chunks)
                def _start():
                    pltpu.async_copy(hp_hbm.at[srcv.at[j + 1]],
                                     rows.at[1 - b], sems[1 - b])

                pltpu.make_async_copy(hp_hbm.at[srcv.at[j]], rows.at[b],
                                      sems[b]).wait()
                pltpu.sync_copy(rows.at[b], acc_sh.at[dstv.at[j]], add=True)
            return carry

        lax.fori_loop(0, n_chunks // 2, outer, 0)
        plsc.subcore_barrier()

        for t in range(n_cp):
            pltpu.sync_copy(acc_sh.at[pl.ds(base + t * _CHUNK, _CHUNK)],
                            out_hbm.at[pl.ds(out_base + t * _CHUNK, _CHUNK)])

    return edge


def _mm1_body(x_ref, w_ref, d0_ref, d1_ref, hp_ref, dinv_ref):
    dinv = lax.rsqrt(d0_ref[...] + d1_ref[...] + 1.0)
    h = jnp.dot(x_ref[...], w_ref[...], preferred_element_type=jnp.float32)
    hp_ref[...] = dinv * h
    dinv_ref[...] = dinv


def _mm2_body(x_ref, w_ref, dinv_ref, hp_ref):
    hp_ref[...] = dinv_ref[...] * jnp.dot(
        x_ref[...], w_ref[...], preferred_element_type=jnp.float32)


def _post_body(al_ref, ah_ref, hp_ref, dv_ref, b_ref, g_ref, bb_ref,
               out_ref, *, final):
    dv = dv_ref[...]
    hpv = hp_ref[...]
    bv = b_ref[...]
    sl = dv[:, 0:_H] * (al_ref[...] + hpv[:, 0:_H]) + bv[:, 0:_H]
    sh = dv[:, _H:_D] * (ah_ref[...] + hpv[:, _H:_D]) + bv[:, _H:_D]
    mu = (jnp.sum(sl, -1, keepdims=True) +
          jnp.sum(sh, -1, keepdims=True)) * (1.0 / _D)
    cl = sl - mu
    ch = sh - mu
    var = (jnp.sum(cl * cl, -1, keepdims=True) +
           jnp.sum(ch * ch, -1, keepdims=True)) * (1.0 / _D)
    r = lax.rsqrt(var + 1e-5)
    gv = g_ref[...]
    bbv = bb_ref[...]
    tl = cl * r * gv[:, 0:_H] + bbv[:, 0:_H]
    th = ch * r * gv[:, _H:_D] + bbv[:, _H:_D]
    el = jnp.where(tl > 0, tl, jnp.exp(jnp.minimum(tl, 0.0)) - 1.0)
    eh = jnp.where(th > 0, th, jnp.exp(jnp.minimum(th, 0.0)) - 1.0)
    if final:
        m = jnp.maximum(jnp.max(el, -1, keepdims=True),
                        jnp.max(eh, -1, keepdims=True))
        lse = jnp.log(jnp.sum(jnp.exp(el - m), -1, keepdims=True) +
                      jnp.sum(jnp.exp(eh - m), -1, keepdims=True)) + m
        el = el - lse
        eh = eh - lse
    out_ref[:, 0:_H] = el
    out_ref[:, _H:_D] = eh


def _row_spec():
    return pl.BlockSpec((_ROW_BLK, _D), lambda i: (i, 0))


@functools.lru_cache(None)
def _mm1_call(n_pad):
    nb = n_pad // _ROW_BLK
    return pl.pallas_call(
        _mm1_body,
        grid=(nb,),
        in_specs=[
            _row_spec(),
            pl.BlockSpec((_D, _D), lambda i: (0, 0)),
            _row_spec(),
            pl.BlockSpec((_ROW_BLK, _D), lambda i: (nb + i, 0)),
        ],
        out_specs=[_row_spec(), _row_spec()],
        out_shape=[jax.ShapeDtypeStruct((n_pad, _D), jnp.float32),
                   jax.ShapeDtypeStruct((n_pad, _D), jnp.float32)],
    )


@functools.lru_cache(None)
def _mm2_call(n_pad):
    nb = n_pad // _ROW_BLK
    return pl.pallas_call(
        _mm2_body,
        grid=(nb,),
        in_specs=[
            _row_spec(),
            pl.BlockSpec((_D, _D), lambda i: (0, 0)),
            _row_spec(),
        ],
        out_specs=_row_spec(),
        out_shape=jax.ShapeDtypeStruct((n_pad, _D), jnp.float32),
    )


@functools.lru_cache(None)
def _post_call(n_pad, final):
    nb = n_pad // _ROW_BLK
    half = pl.BlockSpec((_ROW_BLK, _H), lambda i: (i, 0))
    halfhi = pl.BlockSpec((_ROW_BLK, _H), lambda i: (nb + i, 0))
    vec = pl.BlockSpec((1, _D), lambda i: (0, 0))
    return pl.pallas_call(
        functools.partial(_post_body, final=final),
        grid=(nb,),
        in_specs=[half, halfhi, _row_spec(), _row_spec(), vec, vec, vec],
        out_specs=pl.BlockSpec((_ROW_BLK, _D), lambda i: (i, 0)),
        out_shape=jax.ShapeDtypeStruct((n_pad, _D), jnp.float32),
    )


def _postmm_body(al_ref, ah_ref, hp_ref, dv_ref, b_ref, g_ref, bb_ref,
                 w_ref, hp2_ref):
    dv = dv_ref[...]
    hpv = hp_ref[...]
    bv = b_ref[...]
    sl = dv[:, 0:_H] * (al_ref[...] + hpv[:, 0:_H]) + bv[:, 0:_H]
    sh = dv[:, _H:_D] * (ah_ref[...] + hpv[:, _H:_D]) + bv[:, _H:_D]
    mu = (jnp.sum(sl, -1, keepdims=True) +
          jnp.sum(sh, -1, keepdims=True)) * (1.0 / _D)
    cl = sl - mu
    ch = sh - mu
    var = (jnp.sum(cl * cl, -1, keepdims=True) +
           jnp.sum(ch * ch, -1, keepdims=True)) * (1.0 / _D)
    r = lax.rsqrt(var + 1e-5)
    gv = g_ref[...]
    bbv = bb_ref[...]
    tl = cl * r * gv[:, 0:_H] + bbv[:, 0:_H]
    th = ch * r * gv[:, _H:_D] + bbv[:, _H:_D]
    el = jnp.where(tl > 0, tl, jnp.exp(jnp.minimum(tl, 0.0)) - 1.0)
    eh = jnp.where(th > 0, th, jnp.exp(jnp.minimum(th, 0.0)) - 1.0)
    z = jnp.concatenate([el, eh], axis=-1)
    hp2_ref[...] = dv * jnp.dot(z, w_ref[...],
                                preferred_element_type=jnp.float32)


@functools.lru_cache(None)
def _postmm_call(n_pad):
    nb = n_pad // _ROW_BLK
    half = pl.BlockSpec((_ROW_BLK, _H), lambda i: (i, 0))
    halfhi = pl.BlockSpec((_ROW_BLK, _H), lambda i: (nb + i, 0))
    vec = pl.BlockSpec((1, _D), lambda i: (0, 0))
    return pl.pallas_call(
        _postmm_body,
        grid=(nb,),
        in_specs=[half, halfhi, _row_spec(), _row_spec(), vec, vec, vec,
                  pl.BlockSpec((_D, _D), lambda i: (0, 0))],
        out_specs=_row_spec(),
        out_shape=jax.ShapeDtypeStruct((n_pad, _D), jnp.float32),
    )


def kernel(x, edge_index, W1, b1, ln1_g, ln1_b, W2, b2, ln2_g, ln2_b):
    n, d = x.shape
    e = edge_index.shape[1]
    n_pad = _round_up(n, _NS * _CHUNK)
    e_pad = _round_up(e, _NS * _CHUNK * 2)
    n_chunks = e_pad // (_NS * _CHUNK)
    pad = e_pad - e

    padv = jnp.full((pad,), n, jnp.int32)
    src = jnp.concatenate([edge_index[0], padv]).reshape(_NS, n_chunks, _CHUNK)
    dst = jnp.concatenate([edge_index[1], padv]).reshape(_NS, n_chunks, _CHUNK)
    x_pad = jnp.concatenate([x, jnp.zeros((n_pad - n, d), x.dtype)])

    deg = _deg_kernel(n_pad, n_chunks)(dst)

    b1r = b1.reshape(1, _D)
    g1r = ln1_g.reshape(1, _D)
    bb1r = ln1_b.reshape(1, _D)
    b2r = b2.reshape(1, _D)
    g2r = ln2_g.reshape(1, _D)
    bb2r = ln2_b.reshape(1, _D)

    edge_call = _edge_kernel(n_pad, n_chunks)

    hp1, dinvw = _mm1_call(n_pad)(x_pad, W1, deg, deg)
    acc1 = edge_call(src, dst, hp1.reshape(2 * n_pad, _H))
    hp2 = _postmm_call(n_pad)(acc1, acc1, hp1, dinvw, b1r, g1r, bb1r, W2)
    acc2 = edge_call(src, dst, hp2.reshape(2 * n_pad, _H))
    z2 = _post_call(n_pad, True)(acc2, acc2, hp2, dinvw, b2r, g2r, bb2r)
    return z2[:n]
